# baseline (speedup 1.0000x reference)
import functools
import math

import jax
import jax.numpy as jnp
from jax.experimental import pallas as pl
from jax.experimental.pallas import tpu as pltpu

_MOMENTUM = 0.99
_NEW_RATE = 0.01
_NEG_INF = float("-inf")


def _stats_kernel(keys_ref, mem_ref, m_ref, s_ref, m_sc, s_sc):
    i = pl.program_id(0)

    @pl.when(i == 0)
    def _init():
        m_sc[...] = jnp.full_like(m_sc, _NEG_INF)
        s_sc[...] = jnp.zeros_like(s_sc)

    sc = jnp.dot(keys_ref[...], mem_ref[...].T, preferred_element_type=jnp.float32)
    bm = jnp.max(sc, axis=1, keepdims=True)
    m_old = m_sc[...]
    m_new = jnp.maximum(m_old, bm)
    s_sc[...] = s_sc[...] * jnp.exp(m_old - m_new) + jnp.sum(
        jnp.exp(sc - m_new), axis=1, keepdims=True
    )
    m_sc[...] = m_new

    @pl.when(i == pl.num_programs(0) - 1)
    def _fin():
        m_ref[...] = m_sc[...]
        s_ref[...] = s_sc[...]


def _prio_topk_kernel(b, k, nchunk, keys_ref, mem_ref, m_ref, s_ref, imp_ref,
                      idx_ref, ps, cms):
    i = pl.program_id(0)
    sc = jnp.dot(keys_ref[...], mem_ref[...].T, preferred_element_type=jnp.float32)
    e = jnp.exp(sc - m_ref[...]) / s_ref[...]
    cb = sc.shape[1]
    krows = sc.shape[0] // b
    usage = e.reshape(b, krows, cb).sum(axis=1)
    combined = jnp.mean(imp_ref[...], axis=1)
    prio = combined + 0.1 * usage
    nc_blk = cb // 128
    p3 = prio.reshape(b, nc_blk, 128)
    ps[:, pl.ds(i * nc_blk, nc_blk), :] = p3

    @pl.when(i == pl.num_programs(0) - 1)
    def _topk():
        cms[...] = jnp.max(ps[...], axis=2)
        iota_c = jax.lax.broadcasted_iota(jnp.int32, (b, nchunk), 1)
        iota_c1 = jax.lax.broadcasted_iota(jnp.int32, (1, nchunk), 1)
        iota_r = jax.lax.broadcasted_iota(jnp.int32, (1, 128), 1)

        def step(r, carry):
            cm = cms[...]
            mrow = jnp.max(cm, axis=1, keepdims=True)
            c4 = jnp.min(jnp.where(cm == mrow, iota_c, nchunk), axis=1)
            for bb in range(b):
                c = c4[bb]
                row = ps[bb, pl.ds(c, 1), :]
                pos = jnp.argmax(row)
                idxv = (c * 128 + pos).astype(jnp.int32)
                old = idx_ref[pl.ds(bb, 1), :]
                idx_ref[pl.ds(bb, 1), :] = jnp.where(iota_r == r, idxv, old)
                nrow = jnp.where(iota_r == pos, _NEG_INF, row)
                ps[bb, pl.ds(c, 1), :] = nrow
                cmrow = cms[pl.ds(bb, 1), :]
                cms[pl.ds(bb, 1), :] = jnp.where(iota_c1 == c, jnp.max(nrow), cmrow)
            return carry

        jax.lax.fori_loop(0, k, step, 0)


def _scatter_kernel(b, k, scale, idx_ref, mem_ref, vals_ref, out_ref):
    out_ref[...] = mem_ref[...] * _MOMENTUM

    for bb in range(b):
        def body(i, carry, bb=bb):
            j = idx_ref[bb, i]
            out_ref[pl.ds(j, 1), :] = (
                out_ref[pl.ds(j, 1), :] + vals_ref[bb, pl.ds(i, 1), :] * scale
            )
            return carry

        jax.lax.fori_loop(0, k, body, 0)


def _read_stats_kernel(inv_sqrt_d, q_ref, kv_ref, m_ref, s_ref, m_sc, s_sc):
    i = pl.program_id(0)

    @pl.when(i == 0)
    def _init():
        m_sc[...] = jnp.full_like(m_sc, _NEG_INF)
        s_sc[...] = jnp.zeros_like(s_sc)

    l = jnp.dot(q_ref[...], kv_ref[...].T, preferred_element_type=jnp.float32) * inv_sqrt_d
    bm = jnp.max(l, axis=1, keepdims=True)
    m_old = m_sc[...]
    m_new = jnp.maximum(m_old, bm)
    s_sc[...] = s_sc[...] * jnp.exp(m_old - m_new) + jnp.sum(
        jnp.exp(l - m_new), axis=1, keepdims=True
    )
    m_sc[...] = m_new

    @pl.when(i == pl.num_programs(0) - 1)
    def _fin():
        m_ref[...] = m_sc[...]
        s_ref[...] = s_sc[...]


def _read_out_kernel(inv_sqrt_d, q_ref, kv_ref, m_ref, s_ref, o_ref, acc):
    i = pl.program_id(0)

    @pl.when(i == 0)
    def _init():
        acc[...] = jnp.zeros_like(acc)

    l = jnp.dot(q_ref[...], kv_ref[...].T, preferred_element_type=jnp.float32) * inv_sqrt_d
    w = jnp.exp(l - m_ref[...]) / s_ref[...]
    acc[...] += jnp.dot(w, kv_ref[...], preferred_element_type=jnp.float32)

    @pl.when(i == pl.num_programs(0) - 1)
    def _fin():
        o_ref[...] = acc[...]


def kernel(keys, values, importance, query, mem):
    b, k_orig, d = keys.shape
    cap = mem.shape[1]
    q = query.shape[1]
    k = min(k_orig, cap)
    bk = b * k_orig
    bq = b * q

    mem2 = mem.reshape(cap, d)
    keys2 = keys.reshape(bk, d)
    imp2 = importance.reshape(b, -1, cap)
    nplane = imp2.shape[1]
    query2 = query.reshape(bq, d)
    values_w = values if k == k_orig else values[:, :k]

    cb = 2048
    grid = cap // cb

    m_rows, s_rows = pl.pallas_call(
        _stats_kernel,
        grid=(grid,),
        in_specs=[
            pl.BlockSpec((bk, d), lambda i: (0, 0)),
            pl.BlockSpec((cb, d), lambda i: (i, 0)),
        ],
        out_specs=[
            pl.BlockSpec((bk, 1), lambda i: (0, 0)),
            pl.BlockSpec((bk, 1), lambda i: (0, 0)),
        ],
        out_shape=[
            jax.ShapeDtypeStruct((bk, 1), jnp.float32),
            jax.ShapeDtypeStruct((bk, 1), jnp.float32),
        ],
        scratch_shapes=[
            pltpu.VMEM((bk, 1), jnp.float32),
            pltpu.VMEM((bk, 1), jnp.float32),
        ],
    )(keys2, mem2)

    nchunk = cap // 128
    indices = pl.pallas_call(
        functools.partial(_prio_topk_kernel, b, k, nchunk),
        grid=(grid,),
        in_specs=[
            pl.BlockSpec((bk, d), lambda i: (0, 0)),
            pl.BlockSpec((cb, d), lambda i: (i, 0)),
            pl.BlockSpec((bk, 1), lambda i: (0, 0)),
            pl.BlockSpec((bk, 1), lambda i: (0, 0)),
            pl.BlockSpec((b, nplane, cb), lambda i: (0, 0, i)),
        ],
        out_specs=pl.BlockSpec((b, 128), lambda i: (0, 0)),
        out_shape=jax.ShapeDtypeStruct((b, 128), jnp.int32),
        scratch_shapes=[
            pltpu.VMEM((b, nchunk, 128), jnp.float32),
            pltpu.VMEM((b, nchunk), jnp.float32),
        ],
    )(keys2, mem2, m_rows, s_rows, imp2)

    scale = _NEW_RATE / b
    new_mem = pl.pallas_call(
        functools.partial(_scatter_kernel, b, k, scale),
        in_specs=[
            pl.BlockSpec(memory_space=pltpu.SMEM),
            pl.BlockSpec((cap, d), lambda: (0, 0)),
            pl.BlockSpec((b, k, d), lambda: (0, 0, 0)),
        ],
        out_specs=pl.BlockSpec((cap, d), lambda: (0, 0)),
        out_shape=jax.ShapeDtypeStruct((cap, d), jnp.float32),
    )(indices, mem2, values_w)

    fb = 1024
    rgrid = cap // fb
    isd = 1.0 / math.sqrt(d)
    m_q, s_q = pl.pallas_call(
        functools.partial(_read_stats_kernel, isd),
        grid=(rgrid,),
        in_specs=[
            pl.BlockSpec((bq, d), lambda i: (0, 0)),
            pl.BlockSpec((fb, d), lambda i: (i, 0)),
        ],
        out_specs=[
            pl.BlockSpec((bq, 1), lambda i: (0, 0)),
            pl.BlockSpec((bq, 1), lambda i: (0, 0)),
        ],
        out_shape=[
            jax.ShapeDtypeStruct((bq, 1), jnp.float32),
            jax.ShapeDtypeStruct((bq, 1), jnp.float32),
        ],
        scratch_shapes=[
            pltpu.VMEM((bq, 1), jnp.float32),
            pltpu.VMEM((bq, 1), jnp.float32),
        ],
    )(query2, new_mem)

    out = pl.pallas_call(
        functools.partial(_read_out_kernel, isd),
        grid=(rgrid,),
        in_specs=[
            pl.BlockSpec((bq, d), lambda i: (0, 0)),
            pl.BlockSpec((fb, d), lambda i: (i, 0)),
            pl.BlockSpec((bq, 1), lambda i: (0, 0)),
            pl.BlockSpec((bq, 1), lambda i: (0, 0)),
        ],
        out_specs=pl.BlockSpec((bq, d), lambda i: (0, 0)),
        out_shape=jax.ShapeDtypeStruct((bq, d), jnp.float32),
        scratch_shapes=[pltpu.VMEM((bq, d), jnp.float32)],
    )(query2, new_mem, m_q, s_q)

    return out.reshape(b, q, d)


# speedup vs baseline: 1.2195x; 1.2195x over previous
import functools
import math

import jax
import jax.numpy as jnp
from jax.experimental import pallas as pl
from jax.experimental.pallas import tpu as pltpu

_MOMENTUM = 0.99
_NEW_RATE = 0.01
_NEG_INF = float("-inf")


def _stats_kernel(keys_ref, mem_ref, m_ref, s_ref, m_sc, s_sc):
    i = pl.program_id(0)

    @pl.when(i == 0)
    def _init():
        m_sc[...] = jnp.full_like(m_sc, _NEG_INF)
        s_sc[...] = jnp.zeros_like(s_sc)

    sc = jnp.dot(keys_ref[...], mem_ref[...].T, preferred_element_type=jnp.float32)
    bm = jnp.max(sc, axis=1, keepdims=True)
    m_old = m_sc[...]
    m_new = jnp.maximum(m_old, bm)
    s_sc[...] = s_sc[...] * jnp.exp(m_old - m_new) + jnp.sum(
        jnp.exp(sc - m_new), axis=1, keepdims=True
    )
    m_sc[...] = m_new

    @pl.when(i == pl.num_programs(0) - 1)
    def _fin():
        m_ref[...] = m_sc[...]
        s_ref[...] = s_sc[...]


def _prio_topk_kernel(b, k, nchunk, keys_ref, mem_ref, m_ref, s_ref, imp_ref,
                      idx_ref, ps):
    i = pl.program_id(0)
    sc = jnp.dot(keys_ref[...], mem_ref[...].T, preferred_element_type=jnp.float32)
    e = jnp.exp(sc - m_ref[...]) / s_ref[...]
    cb = sc.shape[1]
    krows = sc.shape[0] // b
    usage = e.reshape(b, krows, cb).sum(axis=1)
    combined = jnp.mean(imp_ref[...], axis=1)
    prio = combined + 0.1 * usage
    nc_blk = cb // 128
    p3 = prio.reshape(b, nc_blk, 128)
    ps[:, pl.ds(i * nc_blk, nc_blk), :] = p3

    @pl.when(i == pl.num_programs(0) - 1)
    def _topk():
        pall = ps[...]
        bits = jax.lax.bitcast_convert_type(pall, jnp.int32)

        def bs_step(_, carry):
            lo, hi = carry
            mid = lo + jax.lax.shift_right_logical(hi - lo, 1)
            gt = (bits > mid[:, :, None]).astype(jnp.float32)
            cnt = jnp.sum(jnp.sum(gt, axis=2), axis=1, keepdims=True)
            pred = cnt >= k
            lo = jnp.where(pred, mid + 1, lo)
            hi = jnp.where(pred, hi, mid)
            return lo, hi

        lo0 = jnp.zeros((b, 1), jnp.int32)
        hi0 = jnp.full((b, 1), 0x7F800000, jnp.int32)
        _, t_bits = jax.lax.fori_loop(0, 31, bs_step, (lo0, hi0))
        tb = t_bits[:, :, None]
        mask_h = bits > tb
        mask_e = bits == tb

        io128a = jax.lax.broadcasted_iota(jnp.int32, (128, 128), 0)
        io128b = jax.lax.broadcasted_iota(jnp.int32, (128, 128), 1)
        m_incl = (io128a <= io128b).astype(jnp.float32)
        ioca = jax.lax.broadcasted_iota(jnp.int32, (nchunk, nchunk), 0)
        iocb = jax.lax.broadcasted_iota(jnp.int32, (nchunk, nchunk), 1)
        m_strict = (ioca < iocb).astype(jnp.float32)

        def slots(mask):
            f = mask.astype(jnp.float32)
            cs = jnp.dot(f.reshape(b * nchunk, 128), m_incl,
                         preferred_element_type=jnp.float32).reshape(b, nchunk, 128)
            ccnt = jnp.sum(f, axis=2)
            off = jnp.dot(ccnt, m_strict, preferred_element_type=jnp.float32)
            tot = jnp.sum(ccnt, axis=1, keepdims=True)
            return off[:, :, None] + cs - 1.0, tot

        slot_h, c1 = slots(mask_h)
        pos_e, _ = slots(mask_e)
        keep_e = mask_e & (pos_e < (k - c1)[:, :, None])
        sel = mask_h | keep_e
        slot = jnp.where(mask_h, slot_h, c1[:, :, None] + pos_e)
        slot = jnp.where(sel, slot, -1.0)

        io_r3 = jax.lax.broadcasted_iota(jnp.int32, (1, 1, 128), 2).astype(jnp.float32)
        io_c3 = jax.lax.broadcasted_iota(jnp.int32, (1, nchunk, 1), 1).astype(jnp.float32)
        io_l3 = jax.lax.broadcasted_iota(jnp.int32, (1, 1, 128), 2).astype(jnp.float32)
        jfull = io_c3 * 128.0 + io_l3
        io_k = jax.lax.broadcasted_iota(jnp.int32, (b, 128), 1)

        def ext_step(r, carry):
            cv, ci = carry
            hit = slot == r
            contrib_v = jnp.sum(jnp.sum(jnp.where(hit, pall, 0.0), axis=2), axis=1)
            contrib_i = jnp.sum(jnp.sum(jnp.where(hit, jfull, 0.0), axis=2), axis=1)
            put = io_k == r
            cv = jnp.where(put, contrib_v[:, None], cv)
            ci = jnp.where(put, contrib_i[:, None], ci)
            return cv, ci

        cand_v, cand_i = jax.lax.fori_loop(
            0, k, ext_step,
            (jnp.zeros((b, 128), jnp.float32), jnp.zeros((b, 128), jnp.float32)))

        va = cand_v[:, :, None]
        vb = cand_v[:, None, :]
        io_rp = jax.lax.broadcasted_iota(jnp.int32, (1, 1, 128), 2)
        io_rr = jax.lax.broadcasted_iota(jnp.int32, (1, 128, 1), 1)
        gt = (vb > va).astype(jnp.float32)
        tie = ((vb == va) & (io_rp < io_rr)).astype(jnp.float32)
        rank = jnp.sum(gt + tie, axis=2)

        oh = (rank[:, :, None] == io_r3).astype(jnp.float32)
        out = jnp.sum(oh * cand_i[:, :, None], axis=1)
        idx_ref[...] = out.astype(jnp.int32)


def _scatter_kernel(b, k, cb, idx_ref, mem_ref, vals_ref, out_ref):
    i = pl.program_id(0)
    base = i * cb
    jio = jax.lax.broadcasted_iota(jnp.int32, (cb, 1), 0) + base
    cols = []
    for bb in range(b):
        cols.append((jio == idx_ref[pl.ds(bb, 1), :]).astype(jnp.float32))
    e = jnp.concatenate(cols, axis=1)
    delta = jnp.dot(e, vals_ref[...], preferred_element_type=jnp.float32)
    out_ref[...] = mem_ref[...] * _MOMENTUM + (delta * (1.0 / b)) * _NEW_RATE


def _read_stats_kernel(inv_sqrt_d, q_ref, kv_ref, m_ref, s_ref, m_sc, s_sc):
    i = pl.program_id(0)

    @pl.when(i == 0)
    def _init():
        m_sc[...] = jnp.full_like(m_sc, _NEG_INF)
        s_sc[...] = jnp.zeros_like(s_sc)

    l = jnp.dot(q_ref[...], kv_ref[...].T, preferred_element_type=jnp.float32) * inv_sqrt_d
    bm = jnp.max(l, axis=1, keepdims=True)
    m_old = m_sc[...]
    m_new = jnp.maximum(m_old, bm)
    s_sc[...] = s_sc[...] * jnp.exp(m_old - m_new) + jnp.sum(
        jnp.exp(l - m_new), axis=1, keepdims=True
    )
    m_sc[...] = m_new

    @pl.when(i == pl.num_programs(0) - 1)
    def _fin():
        m_ref[...] = m_sc[...]
        s_ref[...] = s_sc[...]


def _read_out_kernel(inv_sqrt_d, q_ref, kv_ref, m_ref, s_ref, o_ref, acc):
    i = pl.program_id(0)

    @pl.when(i == 0)
    def _init():
        acc[...] = jnp.zeros_like(acc)

    l = jnp.dot(q_ref[...], kv_ref[...].T, preferred_element_type=jnp.float32) * inv_sqrt_d
    w = jnp.exp(l - m_ref[...]) / s_ref[...]
    acc[...] += jnp.dot(w, kv_ref[...], preferred_element_type=jnp.float32)

    @pl.when(i == pl.num_programs(0) - 1)
    def _fin():
        o_ref[...] = acc[...]


def kernel(keys, values, importance, query, mem):
    b, k_orig, d = keys.shape
    cap = mem.shape[1]
    q = query.shape[1]
    k = min(k_orig, cap)
    bk = b * k_orig
    bq = b * q

    mem2 = mem.reshape(cap, d)
    keys2 = keys.reshape(bk, d)
    imp2 = importance.reshape(b, -1, cap)
    nplane = imp2.shape[1]
    query2 = query.reshape(bq, d)
    values_w = values if k == k_orig else values[:, :k]

    cb = 2048
    grid = cap // cb

    m_rows, s_rows = pl.pallas_call(
        _stats_kernel,
        grid=(grid,),
        in_specs=[
            pl.BlockSpec((bk, d), lambda i: (0, 0)),
            pl.BlockSpec((cb, d), lambda i: (i, 0)),
        ],
        out_specs=[
            pl.BlockSpec((bk, 1), lambda i: (0, 0)),
            pl.BlockSpec((bk, 1), lambda i: (0, 0)),
        ],
        out_shape=[
            jax.ShapeDtypeStruct((bk, 1), jnp.float32),
            jax.ShapeDtypeStruct((bk, 1), jnp.float32),
        ],
        scratch_shapes=[
            pltpu.VMEM((bk, 1), jnp.float32),
            pltpu.VMEM((bk, 1), jnp.float32),
        ],
    )(keys2, mem2)

    nchunk = cap // 128
    indices = pl.pallas_call(
        functools.partial(_prio_topk_kernel, b, k, nchunk),
        grid=(grid,),
        in_specs=[
            pl.BlockSpec((bk, d), lambda i: (0, 0)),
            pl.BlockSpec((cb, d), lambda i: (i, 0)),
            pl.BlockSpec((bk, 1), lambda i: (0, 0)),
            pl.BlockSpec((bk, 1), lambda i: (0, 0)),
            pl.BlockSpec((b, nplane, cb), lambda i: (0, 0, i)),
        ],
        out_specs=pl.BlockSpec((b, 128), lambda i: (0, 0)),
        out_shape=jax.ShapeDtypeStruct((b, 128), jnp.int32),
        scratch_shapes=[
            pltpu.VMEM((b, nchunk, 128), jnp.float32),
        ],
    )(keys2, mem2, m_rows, s_rows, imp2)

    vals_flat = values_w.reshape(b * k, d)
    new_mem = pl.pallas_call(
        functools.partial(_scatter_kernel, b, k, cb),
        grid=(grid,),
        in_specs=[
            pl.BlockSpec((b, 128), lambda i: (0, 0)),
            pl.BlockSpec((cb, d), lambda i: (i, 0)),
            pl.BlockSpec((b * k, d), lambda i: (0, 0)),
        ],
        out_specs=pl.BlockSpec((cb, d), lambda i: (i, 0)),
        out_shape=jax.ShapeDtypeStruct((cap, d), jnp.float32),
    )(indices, mem2, vals_flat)

    fb = 1024
    rgrid = cap // fb
    isd = 1.0 / math.sqrt(d)
    m_q, s_q = pl.pallas_call(
        functools.partial(_read_stats_kernel, isd),
        grid=(rgrid,),
        in_specs=[
            pl.BlockSpec((bq, d), lambda i: (0, 0)),
            pl.BlockSpec((fb, d), lambda i: (i, 0)),
        ],
        out_specs=[
            pl.BlockSpec((bq, 1), lambda i: (0, 0)),
            pl.BlockSpec((bq, 1), lambda i: (0, 0)),
        ],
        out_shape=[
            jax.ShapeDtypeStruct((bq, 1), jnp.float32),
            jax.ShapeDtypeStruct((bq, 1), jnp.float32),
        ],
        scratch_shapes=[
            pltpu.VMEM((bq, 1), jnp.float32),
            pltpu.VMEM((bq, 1), jnp.float32),
        ],
    )(query2, new_mem)

    out = pl.pallas_call(
        functools.partial(_read_out_kernel, isd),
        grid=(rgrid,),
        in_specs=[
            pl.BlockSpec((bq, d), lambda i: (0, 0)),
            pl.BlockSpec((fb, d), lambda i: (i, 0)),
            pl.BlockSpec((bq, 1), lambda i: (0, 0)),
            pl.BlockSpec((bq, 1), lambda i: (0, 0)),
        ],
        out_specs=pl.BlockSpec((bq, d), lambda i: (0, 0)),
        out_shape=jax.ShapeDtypeStruct((bq, d), jnp.float32),
        scratch_shapes=[pltpu.VMEM((bq, d), jnp.float32)],
    )(query2, new_mem, m_q, s_q)

    return out.reshape(b, q, d)
